# baseline (speedup 1.0000x reference)
import functools

import jax
import jax.numpy as jnp
from jax import lax
from jax.experimental import pallas as pl
from jax.experimental.pallas import tpu as pltpu
from jax.experimental.pallas import tpu_sc as plsc

B = 4096
L = 50
D = 64

NC = 2
NS = 16
NW = NC * NS
RPW = B // NW
NB = 4


def _build():
    mesh = plsc.VectorSubcoreMesh(core_axis_name="c", subcore_axis_name="s")

    @functools.partial(
        pl.kernel,
        out_type=jax.ShapeDtypeStruct((B, D), jnp.float32),
        mesh=mesh,
        compiler_params=pltpu.CompilerParams(use_tc_tiling_on_sc=False),
        scratch_types=[
            pltpu.VMEM((RPW, L), jnp.int32),
            pltpu.VMEM((RPW,), jnp.int32),
            pltpu.VMEM((RPW + 16,), jnp.float32),
            pltpu.VMEM((RPW, D), jnp.float32),
        ] + [pltpu.VMEM((L, D), jnp.float32)] * NB
          + [pltpu.SemaphoreType.DMA] * NB,
    )
    def k(kw_h, len_h, table_h, out_h, idx_v, len_v, recip_v, out_v, *rs):
        rbs, sems = rs[:NB], rs[NB:]
        wid = lax.axis_index("s") * NC + lax.axis_index("c")
        row_base = wid * RPW

        pltpu.sync_copy(kw_h.at[pl.ds(row_base, RPW)], idx_v)
        pltpu.sync_copy(len_h.at[pl.ds(row_base, RPW)], len_v)
        for g in range(RPW // 16):
            lv = len_v[pl.ds(g * 16, 16)]
            recip_v[pl.ds(g * 16, 16)] = 1.0 / jnp.maximum(lv, 1).astype(
                jnp.float32)

        def start(p, rb, sem):
            pltpu.async_copy(table_h.at[idx_v.at[p]], rb, sem)

        def wait(p, rb, sem):
            pltpu.make_async_copy(table_h.at[idx_v.at[p]], rb, sem).wait()

        def process(p, rb):
            def lbody(l, accs):
                a0, a1, a2, a3 = accs
                return (
                    a0 + rb[l, pl.ds(0, 16)],
                    a1 + rb[l, pl.ds(16, 16)],
                    a2 + rb[l, pl.ds(32, 16)],
                    a3 + rb[l, pl.ds(48, 16)],
                )

            z = jnp.zeros((16,), jnp.float32)
            accs = lax.fori_loop(0, L, lbody, (z, z, z, z), unroll=10)
            sv = recip_v[pl.ds(p, 16)]
            s0 = sv[0]
            out_v[p, pl.ds(0, 16)] = accs[0] * s0
            out_v[p, pl.ds(16, 16)] = accs[1] * s0
            out_v[p, pl.ds(32, 16)] = accs[2] * s0
            out_v[p, pl.ds(48, 16)] = accs[3] * s0

        for b in range(NB):
            start(b, rbs[b], sems[b])

        def step(s, carry):
            p0 = NB * s
            for b in range(NB):
                wait(p0 + b, rbs[b], sems[b])
                process(p0 + b, rbs[b])
                start(p0 + b + NB, rbs[b], sems[b])
            return carry

        lax.fori_loop(0, RPW // NB - 1, step, 0)
        for b in range(NB):
            p = RPW - NB + b
            wait(p, rbs[b], sems[b])
            process(p, rbs[b])

        pltpu.sync_copy(out_v, out_h.at[pl.ds(row_base, RPW)])

    return k


_sc_kernel = _build()


def kernel(keyword_lists, keyword_lengths, table):
    lens = keyword_lengths.reshape(B)
    return _sc_kernel(keyword_lists, lens, table)
